# baseline (speedup 1.0000x reference)
import jax
import jax.numpy as jnp
from jax import lax
from jax.experimental import pallas as pl
from jax.experimental.pallas import tpu as pltpu
from jax.experimental.pallas import tpu_sc as plsc

H = 16
Q = 2048
K = 2048
NB = 32
L = 16
VLEN = 4112
THRESH = (12, 16, 23, 32, 46, 64, 91)


def _sc_body(tab_hbm, out_hbm, tab_v, v_v, stga, stgb, sema, semb):
    c_ax = lax.axis_index("c")
    s_ax = lax.axis_index("s")
    wid = s_ax * 2 + c_ax
    head = wid // 2
    i0 = (wid % 2) * (Q // 2)

    pltpu.sync_copy(tab_hbm, tab_v)

    def vbody(g, carry):
        t = g * L + lax.iota(jnp.int32, L)
        d = t - (Q - 1)
        n = -d
        one = jnp.full((L,), 1, jnp.int32)
        zero = jnp.full((L,), 0, jnp.int32)
        side = jnp.where(n < 0, one * (NB // 2), zero)
        m = jnp.abs(n)
        big = jnp.full((L,), 8, jnp.int32)
        for th in THRESH:
            big = big + jnp.where(m >= th, one, zero)
        bk = side + jnp.where(m < 8, m, big)
        vals = plsc.load_gather(tab_v, [bk * H + head])
        v_v[pl.ds(g * L, L)] = vals
        return carry

    lax.fori_loop(0, VLEN // L, vbody, 0)

    def fill_and_send(stage, sem, q):
        c = (Q - 1) - i0 - 8 * q
        for r in range(8):
            for k in range(K // 128):
                for u in range(8):
                    j = k * 128 + u * 16
                    stage[r, pl.ds(j, L)] = v_v[pl.ds(c - r + j, L)]
        rowbase = head * Q + i0 + 8 * q
        pltpu.async_copy(
            stage.at[:, :], out_hbm.at[pl.ds(rowbase, 8), :], sem
        )

    def drain(stage, sem):
        pltpu.make_async_copy(
            stage.at[:, :], out_hbm.at[pl.ds(0, 8), :], sem
        ).wait()

    def mbody(m, carry):
        @pl.when(m > 0)
        def _():
            drain(stga, sema)

        fill_and_send(stga, sema, 2 * m)

        @pl.when(m > 0)
        def _():
            drain(stgb, semb)

        fill_and_send(stgb, semb, 2 * m + 1)
        return carry

    lax.fori_loop(0, (Q // 2) // 16, mbody, 0)
    drain(stga, sema)
    drain(stgb, semb)


def kernel(qlen, klen, table):
    mesh = plsc.VectorSubcoreMesh(core_axis_name="c", subcore_axis_name="s")
    run = pl.kernel(
        _sc_body,
        out_type=jax.ShapeDtypeStruct((H * Q, K), jnp.float32),
        mesh=mesh,
        scratch_types=[
            pltpu.VMEM((NB * H,), jnp.float32),
            pltpu.VMEM((VLEN,), jnp.float32),
            pltpu.VMEM((8, K), jnp.float32),
            pltpu.VMEM((8, K), jnp.float32),
            pltpu.SemaphoreType.DMA,
            pltpu.SemaphoreType.DMA,
        ],
        compiler_params=pltpu.CompilerParams(needs_layout_passes=False),
    )
    flat = run(table.reshape(NB * H))
    return flat.reshape(1, H, Q, K)


# speedup vs baseline: 3.5035x; 3.5035x over previous
import jax
import jax.numpy as jnp
from jax import lax
from jax.experimental import pallas as pl
from jax.experimental.pallas import tpu as pltpu
from jax.experimental.pallas import tpu_sc as plsc

H = 16
Q = 2048
K = 2048
NB = 32
L = 16
VLEN = 4112
THRESH = (12, 16, 23, 32, 46, 64, 91)


def _sc_body(tab_hbm, out_hbm, tab_v, v_v, stage, sem):
    c_ax = lax.axis_index("c")
    s_ax = lax.axis_index("s")
    wid = s_ax * 2 + c_ax
    head = wid // 2
    i0 = (wid % 2) * (Q // 2)

    pltpu.sync_copy(tab_hbm, tab_v)

    def vbody(g, carry):
        t = g * L + lax.iota(jnp.int32, L)
        d = t - (Q - 1)
        n = -d
        one = jnp.full((L,), 1, jnp.int32)
        zero = jnp.full((L,), 0, jnp.int32)
        side = jnp.where(n < 0, one * (NB // 2), zero)
        m = jnp.abs(n)
        big = jnp.full((L,), 8, jnp.int32)
        for th in THRESH:
            big = big + jnp.where(m >= th, one, zero)
        bk = side + jnp.where(m < 8, m, big)
        vals = plsc.load_gather(tab_v, [bk * H + head])
        v_v[pl.ds(g * L, L)] = vals
        return carry

    lax.fori_loop(0, VLEN // L, vbody, 0)

    def fill_tile(pos, lo):
        for r in range(8):
            for u in range(8):
                stage[r, pl.ds(pos * 128 + u * L, L)] = v_v[
                    pl.ds(lo - r + u * L, L)
                ]

    def drain8():
        for _ in range(8):
            pltpu.make_async_copy(
                stage.at[:, pl.ds(0, K)], out_hbm.at[pl.ds(0, 8), :], sem
            ).wait()

    def chain(ch, carry):
        a = (Q - 1) - i0 - 8 * ch

        @pl.when(ch > 0)
        def _():
            drain8()

        for mm in range(16):
            fill_tile(7 + mm, a + 128 * mm)
        for k in range(8):
            if k > 0:
                fill_tile(7 - k, a - 128 * k)
            rowbase = head * Q + i0 + 8 * ch + 128 * k
            pltpu.async_copy(
                stage.at[:, pl.ds((7 - k) * 128, K)],
                out_hbm.at[pl.ds(rowbase, 8), :],
                sem,
            )
        return carry

    lax.fori_loop(0, 16, chain, 0)
    drain8()


def kernel(qlen, klen, table):
    mesh = plsc.VectorSubcoreMesh(core_axis_name="c", subcore_axis_name="s")
    run = pl.kernel(
        _sc_body,
        out_type=jax.ShapeDtypeStruct((H * Q, K), jnp.float32),
        mesh=mesh,
        scratch_types=[
            pltpu.VMEM((NB * H,), jnp.float32),
            pltpu.VMEM((VLEN,), jnp.float32),
            pltpu.VMEM((8, 23 * 128), jnp.float32),
            pltpu.SemaphoreType.DMA,
        ],
        compiler_params=pltpu.CompilerParams(needs_layout_passes=False),
    )
    flat = run(table.reshape(NB * H))
    return flat.reshape(1, H, Q, K)


# speedup vs baseline: 4.1419x; 1.1822x over previous
import jax
import jax.numpy as jnp
from jax import lax
from jax.experimental import pallas as pl
from jax.experimental.pallas import tpu as pltpu
from jax.experimental.pallas import tpu_sc as plsc

H = 16
Q = 2048
K = 2048
NB = 32
L = 16
VLEN = 4112
THRESH = (12, 16, 23, 32, 46, 64, 91)


def _sc_body(tab_hbm, out_hbm, tab_v, v_v, stage, sem):
    c_ax = lax.axis_index("c")
    s_ax = lax.axis_index("s")
    wid = s_ax * 2 + c_ax
    head = wid // 2
    i0 = (wid % 2) * (Q // 2)

    pltpu.sync_copy(tab_hbm, tab_v)

    def vbody(g, carry):
        t = g * L + lax.iota(jnp.int32, L)
        d = t - (Q - 1)
        n = -d
        one = jnp.full((L,), 1, jnp.int32)
        zero = jnp.full((L,), 0, jnp.int32)
        side = jnp.where(n < 0, one * (NB // 2), zero)
        m = jnp.abs(n)
        big = jnp.full((L,), 8, jnp.int32)
        for th in THRESH:
            big = big + jnp.where(m >= th, one, zero)
        bk = side + jnp.where(m < 8, m, big)
        vals = plsc.load_gather(tab_v, [bk * H + head])
        v_v[pl.ds(g * L, L)] = vals
        return carry

    lax.fori_loop(0, VLEN // L, vbody, 0)

    def fill_tile(pos, lo):
        for r in range(8):
            for u in range(8):
                stage[r, pl.ds(pos * 128 + u * L, L)] = v_v[
                    pl.ds(lo - r + u * L, L)
                ]

    def drain8():
        for _ in range(8):
            pltpu.make_async_copy(
                stage.at[:, pl.ds(0, K)], out_hbm.at[pl.ds(0, 8), :], sem
            ).wait()

    def drain1():
        pltpu.make_async_copy(
            stage.at[:, pl.ds(0, K)], out_hbm.at[pl.ds(0, 8), :], sem
        ).wait()

    def chain(ch, carry):
        a = (Q - 1) - i0 - 8 * ch

        for j in range(8):
            @pl.when(ch > 0)
            def _():
                drain1()

            fill_tile(22 - j, a + 128 * (15 - j))
        for mm in range(8):
            fill_tile(14 - mm, a + 128 * (7 - mm))
        for k in range(8):
            if k > 0:
                fill_tile(7 - k, a - 128 * k)
            rowbase = head * Q + i0 + 8 * ch + 128 * k
            pltpu.async_copy(
                stage.at[:, pl.ds((7 - k) * 128, K)],
                out_hbm.at[pl.ds(rowbase, 8), :],
                sem,
            )
        return carry

    lax.fori_loop(0, 16, chain, 0)
    drain8()


def kernel(qlen, klen, table):
    mesh = plsc.VectorSubcoreMesh(core_axis_name="c", subcore_axis_name="s")
    run = pl.kernel(
        _sc_body,
        out_type=jax.ShapeDtypeStruct((H * Q, K), jnp.float32),
        mesh=mesh,
        scratch_types=[
            pltpu.VMEM((NB * H,), jnp.float32),
            pltpu.VMEM((VLEN,), jnp.float32),
            pltpu.VMEM((8, 23 * 128), jnp.float32),
            pltpu.SemaphoreType.DMA,
        ],
        compiler_params=pltpu.CompilerParams(needs_layout_passes=False),
    )
    flat = run(table.reshape(NB * H))
    return flat.reshape(1, H, Q, K)


# speedup vs baseline: 4.5837x; 1.1067x over previous
import jax
import jax.numpy as jnp
from jax import lax
from jax.experimental import pallas as pl
from jax.experimental.pallas import tpu as pltpu
from jax.experimental.pallas import tpu_sc as plsc

H = 16
Q = 2048
K = 2048
NB = 32
L = 16
VLEN = 4112
THRESH = (12, 16, 23, 32, 46, 64, 91)


def _sc_body(tab_hbm, out_hbm, tab_v, v_v, stage, sem):
    c_ax = lax.axis_index("c")
    s_ax = lax.axis_index("s")
    wid = s_ax * 2 + c_ax
    head = wid // 2
    i0 = (wid % 2) * (Q // 2)

    pltpu.sync_copy(tab_hbm, tab_v)

    def vbody(g, carry):
        t = g * L + lax.iota(jnp.int32, L)
        d = t - (Q - 1)
        n = -d
        one = jnp.full((L,), 1, jnp.int32)
        zero = jnp.full((L,), 0, jnp.int32)
        side = jnp.where(n < 0, one * (NB // 2), zero)
        m = jnp.abs(n)
        big = jnp.full((L,), 8, jnp.int32)
        for th in THRESH:
            big = big + jnp.where(m >= th, one, zero)
        bk = side + jnp.where(m < 8, m, big)
        vals = plsc.load_gather(tab_v, [bk * H + head])
        v_v[pl.ds(g * L, L)] = vals
        return carry

    lo_val = plsc.load_gather(tab_v, [jnp.full((L,), 15 * H + head,
                                               jnp.int32)])
    hi_val = plsc.load_gather(tab_v, [jnp.full((L,), 31 * H + head,
                                               jnp.int32)])

    def splat_lo(g, carry):
        v_v[pl.ds(g * L, L)] = lo_val
        return carry

    def splat_hi(g, carry):
        v_v[pl.ds(g * L, L)] = hi_val
        return carry

    lax.fori_loop(0, 122, splat_lo, 0)
    lax.fori_loop(122, 134, vbody, 0)
    lax.fori_loop(134, VLEN // L, splat_hi, 0)

    def fill_tile(pos, lo):
        for r in range(8):
            vals = [v_v[pl.ds(lo - r + u * L, L)] for u in range(8)]
            for u in range(8):
                stage[r, pl.ds(pos * 128 + u * L, L)] = vals[u]

    def drain8():
        for _ in range(8):
            pltpu.make_async_copy(
                stage.at[:, pl.ds(0, K)], out_hbm.at[pl.ds(0, 8), :], sem
            ).wait()

    def drain1():
        pltpu.make_async_copy(
            stage.at[:, pl.ds(0, K)], out_hbm.at[pl.ds(0, 8), :], sem
        ).wait()

    def chain(ch, carry):
        a = (Q - 1) - i0 - 8 * ch

        for j in range(8):
            @pl.when(ch > 0)
            def _():
                drain1()

            fill_tile(22 - j, a + 128 * (15 - j))
        for mm in range(8):
            fill_tile(14 - mm, a + 128 * (7 - mm))
        for k in range(8):
            if k > 0:
                fill_tile(7 - k, a - 128 * k)
            rowbase = head * Q + i0 + 8 * ch + 128 * k
            pltpu.async_copy(
                stage.at[:, pl.ds((7 - k) * 128, K)],
                out_hbm.at[pl.ds(rowbase, 8), :],
                sem,
            )
        return carry

    lax.fori_loop(0, 16, chain, 0)
    drain8()


def kernel(qlen, klen, table):
    mesh = plsc.VectorSubcoreMesh(core_axis_name="c", subcore_axis_name="s")
    run = pl.kernel(
        _sc_body,
        out_type=jax.ShapeDtypeStruct((H * Q, K), jnp.float32),
        mesh=mesh,
        scratch_types=[
            pltpu.VMEM((NB * H,), jnp.float32),
            pltpu.VMEM((VLEN,), jnp.float32),
            pltpu.VMEM((8, 23 * 128), jnp.float32),
            pltpu.SemaphoreType.DMA,
        ],
        compiler_params=pltpu.CompilerParams(needs_layout_passes=False),
    )
    flat = run(table.reshape(NB * H))
    return flat.reshape(1, H, Q, K)


# speedup vs baseline: 4.6809x; 1.0212x over previous
import jax
import jax.numpy as jnp
from jax import lax
from jax.experimental import pallas as pl
from jax.experimental.pallas import tpu as pltpu
from jax.experimental.pallas import tpu_sc as plsc

H = 16
Q = 2048
K = 2048
NB = 32
L = 16
VLEN = 4112
THRESH = (12, 16, 23, 32, 46, 64, 91)


def _sc_body(tab_hbm, out_hbm, tab_v, v_v, stga, stgb, sema, semb):
    c_ax = lax.axis_index("c")
    s_ax = lax.axis_index("s")
    wid = s_ax * 2 + c_ax
    head = wid // 2
    i0 = (wid % 2) * (Q // 2)

    pltpu.sync_copy(tab_hbm, tab_v)

    def vbody(g, carry):
        t = g * L + lax.iota(jnp.int32, L)
        d = t - (Q - 1)
        n = -d
        one = jnp.full((L,), 1, jnp.int32)
        zero = jnp.full((L,), 0, jnp.int32)
        side = jnp.where(n < 0, one * (NB // 2), zero)
        m = jnp.abs(n)
        big = jnp.full((L,), 8, jnp.int32)
        for th in THRESH:
            big = big + jnp.where(m >= th, one, zero)
        bk = side + jnp.where(m < 8, m, big)
        vals = plsc.load_gather(tab_v, [bk * H + head])
        v_v[pl.ds(g * L, L)] = vals
        return carry

    lo_val = plsc.load_gather(tab_v, [jnp.full((L,), 15 * H + head,
                                               jnp.int32)])
    hi_val = plsc.load_gather(tab_v, [jnp.full((L,), 31 * H + head,
                                               jnp.int32)])

    def splat_lo(g, carry):
        v_v[pl.ds(g * L, L)] = lo_val
        return carry

    def splat_hi(g, carry):
        v_v[pl.ds(g * L, L)] = hi_val
        return carry

    lax.fori_loop(0, 122, splat_lo, 0)
    lax.fori_loop(122, 134, vbody, 0)
    lax.fori_loop(134, VLEN // L, splat_hi, 0)

    def fill_tile(stage, pos, lo):
        for r in range(8):
            vals = [v_v[pl.ds(lo - r + u * L, L)] for u in range(8)]
            for u in range(8):
                stage[r, pl.ds(pos * 128 + u * L, L)] = vals[u]

    def drain1(stage, sem):
        pltpu.make_async_copy(
            stage.at[:, pl.ds(0, K)], out_hbm.at[pl.ds(0, 8), :], sem
        ).wait()

    def chain(stage, sem, ch):
        a = (Q - 1) - i0 - 8 * ch

        for j in range(8):
            @pl.when(ch > 1)
            def _():
                drain1(stage, sem)

            fill_tile(stage, 22 - j, a + 128 * (15 - j))
        for mm in range(8):
            fill_tile(stage, 14 - mm, a + 128 * (7 - mm))
        for k in range(8):
            if k > 0:
                fill_tile(stage, 7 - k, a - 128 * k)
            rowbase = head * Q + i0 + 8 * ch + 128 * k
            pltpu.async_copy(
                stage.at[:, pl.ds((7 - k) * 128, K)],
                out_hbm.at[pl.ds(rowbase, 8), :],
                sem,
            )

    def dchain(m, carry):
        chain(stga, sema, 2 * m)
        chain(stgb, semb, 2 * m + 1)
        return carry

    lax.fori_loop(0, 8, dchain, 0)
    for _ in range(8):
        drain1(stga, sema)
    for _ in range(8):
        drain1(stgb, semb)


def kernel(qlen, klen, table):
    mesh = plsc.VectorSubcoreMesh(core_axis_name="c", subcore_axis_name="s")
    run = pl.kernel(
        _sc_body,
        out_type=jax.ShapeDtypeStruct((H * Q, K), jnp.float32),
        mesh=mesh,
        scratch_types=[
            pltpu.VMEM((NB * H,), jnp.float32),
            pltpu.VMEM((VLEN,), jnp.float32),
            pltpu.VMEM((8, 23 * 128), jnp.float32),
            pltpu.VMEM((8, 23 * 128), jnp.float32),
            pltpu.SemaphoreType.DMA,
            pltpu.SemaphoreType.DMA,
        ],
        compiler_params=pltpu.CompilerParams(needs_layout_passes=False),
    )
    flat = run(table.reshape(NB * H))
    return flat.reshape(1, H, Q, K)
